# baseline (speedup 1.0000x reference)
import dataclasses
import functools

import jax
import jax.numpy as jnp
from jax import lax
from jax.experimental import pallas as pl
from jax.experimental.pallas import tpu as pltpu
from jax.experimental.pallas import tpu_sc as plsc

NUM_CORES = 2
NUM_SUBCORES = 16
NW = NUM_CORES * NUM_SUBCORES
L = 16
D = 64
B = 16384
BPW = B // NW
AUG = 128


def _sc_gather_body(bias_lane, ids_hbm, emb_hbm, brow_hbm, out_hbm,
                    ids_v, bri_v, g_v, br_v, aug_v, sem0, sem1):
    wid = lax.axis_index("s") * NUM_CORES + lax.axis_index("c")
    base = wid * BPW

    pltpu.sync_copy(ids_hbm.at[pl.ds(base, BPW)], ids_v)

    ce = pltpu.async_copy(emb_hbm.at[ids_v], g_v, sem0)

    @pl.loop(0, BPW, step=L)
    def _(o):
        bri_v[pl.ds(o, L)] = lax.shift_right_logical(ids_v[pl.ds(o, L)], 4)

    cb = pltpu.async_copy(brow_hbm.at[bri_v], br_v, sem1)
    ce.wait()
    cb.wait()

    iota = lax.iota(jnp.int32, L)
    fifteen = jnp.full((L,), 15, jnp.int32)
    zeros = jnp.zeros((L,), jnp.float32)
    ones = jnp.ones((L,), jnp.float32)

    @pl.loop(0, BPW, step=L)
    def _(g):
        for j in range(L):
            r = g + j
            for k in range(0, D, L):
                aug_v[r, pl.ds(k, L)] = g_v[r, pl.ds(k, L)]
            for k in range(D, AUG, L):
                aug_v[r, pl.ds(k, L)] = zeros
        blane = lax.bitwise_and(ids_v[pl.ds(g, L)], fifteen)
        bias16 = plsc.load_gather(br_v, [g + iota, blane])
        plsc.store_scatter(
            aug_v, [g + iota, jnp.full((L,), bias_lane, jnp.int32)], bias16)
        plsc.store_scatter(
            aug_v, [g + iota, jnp.full((L,), 129 - bias_lane, jnp.int32)],
            ones)

    pltpu.sync_copy(aug_v, out_hbm.at[pl.ds(base, BPW)])


def _tc_dot_body(a_ref, b_ref, o_ref):
    o_ref[...] = jnp.sum(a_ref[...] * b_ref[...], axis=1)


def _make_sc_gather(bias_lane):
    mesh = plsc.VectorSubcoreMesh(core_axis_name="c", subcore_axis_name="s",
                                  num_cores=NUM_CORES,
                                  num_subcores=NUM_SUBCORES)
    cp = pltpu.CompilerParams()
    if "needs_layout_passes" in pltpu.CompilerParams.__dataclass_fields__:
        cp = dataclasses.replace(cp, needs_layout_passes=False)
    if "use_tc_tiling_on_sc" in pltpu.CompilerParams.__dataclass_fields__:
        cp = dataclasses.replace(cp, use_tc_tiling_on_sc=False)
    return pl.kernel(
        functools.partial(_sc_gather_body, bias_lane),
        out_type=jax.ShapeDtypeStruct((B, AUG), jnp.float32),
        mesh=mesh,
        scratch_types=[
            pltpu.VMEM((BPW,), jnp.int32),
            pltpu.VMEM((BPW,), jnp.int32),
            pltpu.VMEM((BPW, D), jnp.float32),
            pltpu.VMEM((BPW, L), jnp.float32),
            pltpu.VMEM((BPW, AUG), jnp.float32),
            pltpu.SemaphoreType.DMA,
            pltpu.SemaphoreType.DMA,
        ],
        compiler_params=cp,
    )


def kernel(user_ids, item_ids, user_emb, item_emb, user_bias, item_bias):
    uid = user_ids.astype(jnp.int32)
    iid = item_ids.astype(jnp.int32)
    nu = user_bias.shape[0]
    ni = item_bias.shape[0]
    ubias_rows = user_bias.reshape(nu // L, L)
    ibias_rows = item_bias.reshape(ni // L, L)

    ue_aug = _make_sc_gather(64)(uid, user_emb, ubias_rows)
    ie_aug = _make_sc_gather(65)(iid, item_emb, ibias_rows)

    tc_block = 512
    return pl.pallas_call(
        _tc_dot_body,
        out_shape=jax.ShapeDtypeStruct((B,), jnp.float32),
        grid=(B // tc_block,),
        in_specs=[
            pl.BlockSpec((tc_block, AUG), lambda i: (i, 0)),
            pl.BlockSpec((tc_block, AUG), lambda i: (i, 0)),
        ],
        out_specs=pl.BlockSpec((tc_block,), lambda i: (i,)),
    )(ue_aug, ie_aug)
